# baseline (speedup 1.0000x reference)
import functools

import jax
import jax.numpy as jnp
from jax import lax
from jax.experimental import pallas as pl
from jax.experimental.pallas import tpu as pltpu
from jax.experimental.pallas import tpu_sc as plsc

_NC = 2
_NS = 16
_NW = _NC * _NS
_K = 2
_L = 1


def _make_kernel(B, S, F):
    rows_per_w = B // _NW
    mesh = plsc.VectorSubcoreMesh(core_axis_name="c", subcore_axis_name="s")

    @functools.partial(
        pl.kernel,
        mesh=mesh,
        out_type=jax.ShapeDtypeStruct((B, S, F), jnp.float32),
        scratch_types=(
            [pltpu.VMEM((1, S, F), jnp.float32) for _ in range(_K)]
            + [pltpu.SemaphoreType.DMA((_K,)), pltpu.SemaphoreType.DMA((_K,))]
        ),
    )
    def k(x_hbm, o_hbm, *scratch):
        bufs = scratch[:_K]
        in_sems, out_sems = scratch[_K], scratch[_K + 1]
        wid = lax.axis_index("s") * _NC + lax.axis_index("c")
        base = wid * rows_per_w

        def in_copy(r):
            s = r % _K
            return pltpu.make_async_copy(
                x_hbm.at[pl.ds(base + r, 1)], bufs[s], in_sems.at[s])

        def out_copy(r):
            s = r % _K
            return pltpu.make_async_copy(
                bufs[s], o_hbm.at[pl.ds(base + r, 1)], out_sems.at[s])

        for r in range(min(_L, rows_per_w)):
            in_copy(r).start()
        for r in range(rows_per_w):
            j = r + _L
            if j < rows_per_w:
                if j - _K >= 0:
                    out_copy(j - _K).wait()
                in_copy(j).start()
            in_copy(r).wait()
            out_copy(r).start()
        for r in range(max(0, rows_per_w - _K), rows_per_w):
            out_copy(r).wait()

    return k


def kernel(X):
    B, S, F = X.shape
    return _make_kernel(B, S, F)(X)
